# baseline (speedup 1.0000x reference)
import functools

import jax
import jax.numpy as jnp
from jax import lax
from jax.experimental import pallas as pl
from jax.experimental.pallas import tpu as pltpu
from jax.experimental.pallas import tpu_sc as plsc


def kernel(x, table):
    seq = x.shape[1]
    emb = table.shape[1]
    total = seq * emb

    info = plsc.get_sparse_core_info()
    nc, ns = info.num_cores, info.num_subcores
    nw = nc * ns
    assert total % nw == 0
    elems_per = total // nw
    assert elems_per % 8 == 0

    flat = table.reshape(-1)

    mesh = plsc.VectorSubcoreMesh(core_axis_name="c", subcore_axis_name="s")

    @functools.partial(
        pl.kernel,
        mesh=mesh,
        out_type=jax.ShapeDtypeStruct((total,), jnp.float32),
    )
    def copy_rows(table_hbm, out_hbm):
        wid = lax.axis_index("s") * nc + lax.axis_index("c")
        base = wid * elems_per
        pltpu.sync_copy(
            table_hbm.at[pl.ds(base, elems_per)],
            out_hbm.at[pl.ds(base, elems_per)],
        )

    return copy_rows(flat).reshape(seq, emb)


# speedup vs baseline: 7.3901x; 7.3901x over previous
import functools

import jax
import jax.numpy as jnp
from jax import lax
from jax.experimental import pallas as pl
from jax.experimental.pallas import tpu as pltpu
from jax.experimental.pallas import tpu_sc as plsc


def kernel(x, table):
    seq = x.shape[1]
    emb = table.shape[1]
    info = plsc.get_sparse_core_info()
    nc, ns = info.num_cores, info.num_subcores
    nw = nc * ns
    assert seq % nw == 0
    rows_per = seq // nw

    mesh = plsc.VectorSubcoreMesh(core_axis_name="c", subcore_axis_name="s")

    @functools.partial(
        pl.kernel,
        mesh=mesh,
        out_type=jax.ShapeDtypeStruct((seq, emb), jnp.float32),
        scratch_types=[
            pltpu.VMEM((rows_per, emb), jnp.float32),
        ],
    )
    def copy_rows(table_hbm, out_hbm, buf):
        wid = lax.axis_index("s") * nc + lax.axis_index("c")
        base = wid * rows_per
        pltpu.sync_copy(table_hbm.at[pl.ds(base, rows_per)], buf)
        pltpu.sync_copy(buf, out_hbm.at[pl.ds(base, rows_per)])

    return copy_rows(table)


# speedup vs baseline: 9.3483x; 1.2650x over previous
import jax
import jax.numpy as jnp
from jax.experimental import pallas as pl


def kernel(x, table):
    seq = x.shape[1]
    emb = table.shape[1]
    block = 512
    assert seq % block == 0

    def body(t_ref, o_ref):
        o_ref[...] = t_ref[...]

    return pl.pallas_call(
        body,
        grid=(seq // block,),
        in_specs=[pl.BlockSpec((block, emb), lambda i: (i, 0))],
        out_specs=pl.BlockSpec((block, emb), lambda i: (i, 0)),
        out_shape=jax.ShapeDtypeStruct((seq, emb), jnp.float32),
    )(table)
